# baseline (speedup 1.0000x reference)
import functools

import jax
import jax.numpy as jnp
from jax import lax
from jax.experimental import pallas as pl
from jax.experimental.pallas import tpu as pltpu
from jax.experimental.pallas import tpu_sc as plsc

EPS = 1e-05
T_ITERS = 4
NC = 2
NS = 16
NW = NC * NS
L = 16


def _oni_norm(w):
    d = w.shape[0]
    zc = w - jnp.mean(w, axis=1, keepdims=True)
    s = lax.dot_general(zc, zc, (((1,), (1,)), ((), ())),
                        preferred_element_type=jnp.float32)
    eye = jnp.eye(d, dtype=jnp.float32)
    s = s + EPS * eye
    norm_s = jnp.sqrt(jnp.sum(s * s))
    s = s / norm_s
    b = eye
    for _ in range(T_ITERS):
        b2 = jnp.dot(b, b, preferred_element_type=jnp.float32)
        b3 = jnp.dot(b2, b, preferred_element_type=jnp.float32)
        b = 1.5 * b - 0.5 * jnp.dot(b3, s, preferred_element_type=jnp.float32)
    return jnp.dot(b, zc, preferred_element_type=jnp.float32) / jnp.sqrt(norm_s)


def _make_sc_kernels(n_pad, d, epw, ch):
    mesh = plsc.VectorSubcoreMesh(core_axis_name="c", subcore_axis_name="s")
    rpt = n_pad // NS
    nch = epw // ch

    @functools.partial(
        pl.kernel,
        out_type=jax.ShapeDtypeStruct((NW, n_pad), jnp.float32),
        mesh=mesh,
        scratch_types=[
            pltpu.VMEM((n_pad,), jnp.float32),
            pltpu.VMEM((epw,), jnp.int32),
        ],
        compiler_params=pltpu.CompilerParams(needs_layout_passes=False),
    )
    def hist_kernel(col_hbm, out_hbm, hist_v, idx_v):
        cid = lax.axis_index("c")
        sid = lax.axis_index("s")
        wid = sid * NC + cid
        zeros16 = jnp.zeros((L,), jnp.float32)

        def zbody(i, _):
            hist_v[pl.ds(i * L, L)] = zeros16
            return 0
        lax.fori_loop(0, n_pad // L, zbody, 0)

        pltpu.sync_copy(col_hbm.at[pl.ds(wid * epw, epw)], idx_v)
        ones16 = jnp.ones((L,), jnp.float32)

        def body(j, _):
            idx = idx_v[pl.ds(j * L, L)]
            plsc.addupdate_scatter(hist_v, [idx], ones16)
            return 0
        lax.fori_loop(0, epw // L, body, 0)
        pltpu.sync_copy(hist_v, out_hbm.at[wid])

    PG = 2
    PI = 4

    @functools.partial(
        pl.kernel,
        out_type=jax.ShapeDtypeStruct((NC, n_pad, d), jnp.float32),
        mesh=mesh,
        scratch_types=[
            pltpu.VMEM_SHARED((n_pad, d), jnp.float32),
            pltpu.VMEM((PI, ch), jnp.int32),
            pltpu.VMEM((nch, ch), jnp.int32),
            pltpu.VMEM((PG, ch, d), jnp.float32),
            pltpu.SemaphoreType.DMA,
            pltpu.SemaphoreType.DMA,
            pltpu.SemaphoreType.DMA,
            pltpu.SemaphoreType.DMA,
            pltpu.SemaphoreType.DMA,
            pltpu.SemaphoreType.DMA,
        ],
    )
    def agg_kernel(xs_hbm, row_hbm, col3_hbm, out_hbm,
                   acc_sh, ri_v, col_v, g_v, sg0, sg1, si0, si1, si2, si3):
        sem_g = (sg0, sg1)
        sem_i = (si0, si1, si2, si3)
        cid = lax.axis_index("c")
        sid = lax.axis_index("s")
        wid = sid * NC + cid
        rbase = sid * rpt
        ebase = wid * epw
        pltpu.sync_copy(xs_hbm.at[pl.ds(rbase, rpt)],
                        acc_sh.at[pl.ds(rbase, rpt)])
        pltpu.sync_copy(col3_hbm.at[wid], col_v)
        plsc.subcore_barrier()

        for j in range(PI):
            pltpu.async_copy(row_hbm.at[pl.ds(ebase + j * ch, ch)],
                             ri_v.at[j], sem_i[j])
        for b in range(PG):
            pltpu.make_async_copy(row_hbm.at[pl.ds(ebase, ch)],
                                  ri_v.at[b], sem_i[b]).wait()
            pltpu.async_copy(xs_hbm.at[ri_v.at[b]], g_v.at[b], sem_g[b])

        @pl.loop(0, nch, step=PI)
        def _(k):
            for b in range(PI):
                kk = k + b
                gb = b % PG
                ib2 = (b + PG) % PI
                pltpu.make_async_copy(xs_hbm.at[ri_v.at[b]], g_v.at[gb],
                                      sem_g[gb]).wait()
                pltpu.sync_copy(g_v.at[gb], acc_sh.at[col_v.at[kk]], add=True)

                @pl.when(kk + PI < nch)
                def _issue_idx():
                    pltpu.async_copy(
                        row_hbm.at[pl.ds(ebase + (kk + PI) * ch, ch)],
                        ri_v.at[b], sem_i[b])

                @pl.when(kk + PG < nch)
                def _issue_gather():
                    pltpu.make_async_copy(row_hbm.at[pl.ds(ebase, ch)],
                                          ri_v.at[ib2], sem_i[ib2]).wait()
                    pltpu.async_copy(xs_hbm.at[ri_v.at[ib2]], g_v.at[gb],
                                     sem_g[gb])

        plsc.subcore_barrier()
        pltpu.sync_copy(acc_sh.at[pl.ds(rbase, rpt)],
                        out_hbm.at[cid, pl.ds(rbase, rpt)])

    return hist_kernel, agg_kernel


def kernel(x, edge_index, weight):
    n, d = x.shape
    e = edge_index.shape[1]

    blk = 640
    n_pad = ((n + blk - 1) // blk) * blk
    ch = 128
    grp = NW * ch * 4
    epw = ((e + grp - 1) // grp) * (ch * 4)
    e_pad = epw * NW

    row = jnp.concatenate(
        [edge_index[0], jnp.zeros((e_pad - e,), jnp.int32)])
    col = jnp.concatenate(
        [edge_index[1], jnp.full((e_pad - e,), n, jnp.int32)])
    col3 = col.reshape(NW, epw // ch, ch)
    x_pad = jnp.pad(x, ((0, n_pad - n), (0, 0)))

    hist_kernel, agg_kernel = _make_sc_kernels(n_pad, d, epw, ch)
    hist = hist_kernel(col)

    grid_a = n_pad // blk

    def tc_a_body(hist_ref, x_ref, w_ref, xs_ref):
        deg = jnp.sum(hist_ref[...], axis=0) + 1.0
        dis = lax.rsqrt(deg)
        w = _oni_norm(w_ref[...])
        xs_ref[...] = dis[:, None] * jnp.dot(
            x_ref[...], w, preferred_element_type=jnp.float32)

    xs = pl.pallas_call(
        tc_a_body,
        grid=(grid_a,),
        in_specs=[
            pl.BlockSpec((NW, blk), lambda i: (0, i)),
            pl.BlockSpec((blk, d), lambda i: (i, 0)),
            pl.BlockSpec((d, d), lambda i: (0, 0)),
        ],
        out_specs=pl.BlockSpec((blk, d), lambda i: (i, 0)),
        out_shape=jax.ShapeDtypeStruct((n_pad, d), jnp.float32),
    )(hist, x_pad, weight)

    parts = agg_kernel(xs, row, col3)

    blkf = 640
    grid_f = (n + blkf - 1) // blkf

    def tc_f_body(hist_ref, p_ref, xs_ref, out_ref):
        deg = jnp.sum(hist_ref[...], axis=0) + 1.0
        dis = lax.rsqrt(deg)
        out_ref[...] = dis[:, None] * (p_ref[0] + p_ref[1] - xs_ref[...])

    out = pl.pallas_call(
        tc_f_body,
        grid=(grid_f,),
        in_specs=[
            pl.BlockSpec((NW, blkf), lambda i: (0, i)),
            pl.BlockSpec((NC, blkf, d), lambda i: (0, i, 0)),
            pl.BlockSpec((blkf, d), lambda i: (i, 0)),
        ],
        out_specs=pl.BlockSpec((blkf, d), lambda i: (i, 0)),
        out_shape=jax.ShapeDtypeStruct((n, d), jnp.float32),
    )(hist, parts, xs)
    return out


# speedup vs baseline: 2.7684x; 2.7684x over previous
import functools

import jax
import jax.numpy as jnp
from jax import lax
from jax.experimental import pallas as pl
from jax.experimental.pallas import tpu as pltpu
from jax.experimental.pallas import tpu_sc as plsc

EPS = 1e-05
T_ITERS = 4
NC = 2
NS = 16
NW = NC * NS
L = 16


def _oni_norm(w):
    d = w.shape[0]
    zc = w - jnp.mean(w, axis=1, keepdims=True)
    s = lax.dot_general(zc, zc, (((1,), (1,)), ((), ())),
                        preferred_element_type=jnp.float32)
    eye = jnp.eye(d, dtype=jnp.float32)
    s = s + EPS * eye
    norm_s = jnp.sqrt(jnp.sum(s * s))
    s = s / norm_s
    b = eye
    for _ in range(T_ITERS):
        b2 = jnp.dot(b, b, preferred_element_type=jnp.float32)
        b3 = jnp.dot(b2, b, preferred_element_type=jnp.float32)
        b = 1.5 * b - 0.5 * jnp.dot(b3, s, preferred_element_type=jnp.float32)
    return jnp.dot(b, zc, preferred_element_type=jnp.float32) / jnp.sqrt(norm_s)


def _make_sc_kernels(n_pad, d, epw, ch):
    mesh = plsc.VectorSubcoreMesh(core_axis_name="c", subcore_axis_name="s")
    rpt = n_pad // NS
    nch = epw // ch

    @functools.partial(
        pl.kernel,
        out_type=jax.ShapeDtypeStruct((NW, n_pad), jnp.float32),
        mesh=mesh,
        scratch_types=[
            pltpu.VMEM((n_pad,), jnp.float32),
            pltpu.VMEM((epw,), jnp.int32),
        ],
        compiler_params=pltpu.CompilerParams(needs_layout_passes=False),
    )
    def hist_kernel(col_hbm, out_hbm, hist_v, idx_v):
        cid = lax.axis_index("c")
        sid = lax.axis_index("s")
        wid = sid * NC + cid
        zeros16 = jnp.zeros((L,), jnp.float32)

        def zbody(i, _):
            hist_v[pl.ds(i * L, L)] = zeros16
            return 0
        lax.fori_loop(0, n_pad // L, zbody, 0)

        pltpu.sync_copy(col_hbm.at[pl.ds(wid * epw, epw)], idx_v)
        ones16 = jnp.ones((L,), jnp.float32)

        def body(j, _):
            idx = idx_v[pl.ds(j * L, L)]
            plsc.addupdate_scatter(hist_v, [idx], ones16)
            return 0
        lax.fori_loop(0, epw // L, body, 0)
        pltpu.sync_copy(hist_v, out_hbm.at[wid])

    PG = 2
    PI = 4

    @functools.partial(
        pl.kernel,
        out_type=jax.ShapeDtypeStruct((NC, n_pad, d), jnp.float32),
        mesh=mesh,
        scratch_types=[
            pltpu.VMEM_SHARED((n_pad, d), jnp.float32),
            pltpu.VMEM((PI, ch), jnp.int32),
            pltpu.VMEM((nch, ch), jnp.int32),
            pltpu.VMEM((PG, ch, d), jnp.float32),
            pltpu.SemaphoreType.DMA,
            pltpu.SemaphoreType.DMA,
            pltpu.SemaphoreType.DMA,
            pltpu.SemaphoreType.DMA,
            pltpu.SemaphoreType.DMA,
            pltpu.SemaphoreType.DMA,
        ],
    )
    def agg_kernel(xs_hbm, row_hbm, col3_hbm, out_hbm,
                   acc_sh, ri_v, col_v, g_v, sg0, sg1, si0, si1, si2, si3):
        sem_g = (sg0, sg1)
        sem_i = (si0, si1, si2, si3)
        cid = lax.axis_index("c")
        sid = lax.axis_index("s")
        wid = sid * NC + cid
        rbase = sid * rpt
        ebase = wid * epw
        pltpu.sync_copy(xs_hbm.at[pl.ds(rbase, rpt)],
                        acc_sh.at[pl.ds(rbase, rpt)])
        pltpu.sync_copy(col3_hbm.at[wid], col_v)
        plsc.subcore_barrier()

        for j in range(PI):
            pltpu.async_copy(row_hbm.at[pl.ds(ebase + j * ch, ch)],
                             ri_v.at[j], sem_i[j])
        for b in range(PG):
            pltpu.make_async_copy(row_hbm.at[pl.ds(ebase, ch)],
                                  ri_v.at[b], sem_i[b]).wait()
            pltpu.async_copy(xs_hbm.at[ri_v.at[b]], g_v.at[b], sem_g[b])

        @pl.loop(0, nch, step=PI)
        def _(k):
            for b in range(PI):
                kk = k + b
                gb = b % PG
                ib2 = (b + PG) % PI
                pltpu.make_async_copy(xs_hbm.at[ri_v.at[b]], g_v.at[gb],
                                      sem_g[gb]).wait()
                pltpu.sync_copy(g_v.at[gb], acc_sh.at[col_v.at[kk]], add=True)

                @pl.when(kk + PI < nch)
                def _issue_idx():
                    pltpu.async_copy(
                        row_hbm.at[pl.ds(ebase + (kk + PI) * ch, ch)],
                        ri_v.at[b], sem_i[b])

                @pl.when(kk + PG < nch)
                def _issue_gather():
                    pltpu.make_async_copy(row_hbm.at[pl.ds(ebase, ch)],
                                          ri_v.at[ib2], sem_i[ib2]).wait()
                    pltpu.async_copy(xs_hbm.at[ri_v.at[ib2]], g_v.at[gb],
                                     sem_g[gb])

        plsc.subcore_barrier()
        pltpu.sync_copy(acc_sh.at[pl.ds(rbase, rpt)],
                        out_hbm.at[cid, pl.ds(rbase, rpt)])

    return hist_kernel, agg_kernel


def kernel(x, edge_index, weight):
    n, d = x.shape
    e = edge_index.shape[1]

    blk = 640
    n_pad = ((n + blk - 1) // blk) * blk
    ch = 128
    grp = NW * ch * 4
    epw = ((e + grp - 1) // grp) * (ch * 4)
    e_pad = epw * NW

    npad_ids = jnp.arange(e_pad - e, dtype=jnp.int32)
    row = jnp.concatenate([edge_index[0], npad_ids % n])
    col = jnp.concatenate([edge_index[1], n + npad_ids % (n_pad - n)])
    col3 = col.reshape(NW, epw // ch, ch)
    x_pad = jnp.pad(x, ((0, n_pad - n), (0, 0)))

    hist_kernel, agg_kernel = _make_sc_kernels(n_pad, d, epw, ch)
    hist = hist_kernel(col)

    grid_a = n_pad // blk

    def tc_a_body(hist_ref, x_ref, w_ref, xs_ref):
        deg = jnp.sum(hist_ref[...], axis=0) + 1.0
        dis = lax.rsqrt(deg)
        w = _oni_norm(w_ref[...])
        xs_ref[...] = dis[:, None] * jnp.dot(
            x_ref[...], w, preferred_element_type=jnp.float32)

    xs = pl.pallas_call(
        tc_a_body,
        grid=(grid_a,),
        in_specs=[
            pl.BlockSpec((NW, blk), lambda i: (0, i)),
            pl.BlockSpec((blk, d), lambda i: (i, 0)),
            pl.BlockSpec((d, d), lambda i: (0, 0)),
        ],
        out_specs=pl.BlockSpec((blk, d), lambda i: (i, 0)),
        out_shape=jax.ShapeDtypeStruct((n_pad, d), jnp.float32),
    )(hist, x_pad, weight)

    parts = agg_kernel(xs, row, col3)

    blkf = 640
    grid_f = (n + blkf - 1) // blkf

    def tc_f_body(hist_ref, p_ref, xs_ref, out_ref):
        deg = jnp.sum(hist_ref[...], axis=0) + 1.0
        dis = lax.rsqrt(deg)
        out_ref[...] = dis[:, None] * (p_ref[0] + p_ref[1] - xs_ref[...])

    out = pl.pallas_call(
        tc_f_body,
        grid=(grid_f,),
        in_specs=[
            pl.BlockSpec((NW, blkf), lambda i: (0, i)),
            pl.BlockSpec((NC, blkf, d), lambda i: (0, i, 0)),
            pl.BlockSpec((blkf, d), lambda i: (i, 0)),
        ],
        out_specs=pl.BlockSpec((blkf, d), lambda i: (i, 0)),
        out_shape=jax.ShapeDtypeStruct((n, d), jnp.float32),
    )(hist, parts, xs)
    return out


# speedup vs baseline: 3.6102x; 1.3041x over previous
import functools

import jax
import jax.numpy as jnp
from jax import lax
from jax.experimental import pallas as pl
from jax.experimental.pallas import tpu as pltpu
from jax.experimental.pallas import tpu_sc as plsc

EPS = 1e-05
T_ITERS = 4
NC = 2
NS = 16
NW = NC * NS
L = 16


def _oni_norm(w):
    d = w.shape[0]
    zc = w - jnp.mean(w, axis=1, keepdims=True)
    s = lax.dot_general(zc, zc, (((1,), (1,)), ((), ())),
                        preferred_element_type=jnp.float32)
    eye = jnp.eye(d, dtype=jnp.float32)
    s = s + EPS * eye
    norm_s = jnp.sqrt(jnp.sum(s * s))
    s = s / norm_s
    b = eye
    for _ in range(T_ITERS):
        b2 = jnp.dot(b, b, preferred_element_type=jnp.float32)
        b3 = jnp.dot(b2, b, preferred_element_type=jnp.float32)
        b = 1.5 * b - 0.5 * jnp.dot(b3, s, preferred_element_type=jnp.float32)
    return jnp.dot(b, zc, preferred_element_type=jnp.float32) / jnp.sqrt(norm_s)


def _make_sc_kernels(n_pad, d, epw, ch):
    mesh = plsc.VectorSubcoreMesh(core_axis_name="c", subcore_axis_name="s")
    rpt = n_pad // NS
    nch = epw // ch

    @functools.partial(
        pl.kernel,
        out_type=jax.ShapeDtypeStruct((NW, n_pad), jnp.float32),
        mesh=mesh,
        scratch_types=[
            pltpu.VMEM((n_pad,), jnp.float32),
            pltpu.VMEM((epw,), jnp.int32),
        ],
        compiler_params=pltpu.CompilerParams(needs_layout_passes=False),
    )
    def hist_kernel(col_hbm, out_hbm, hist_v, idx_v):
        cid = lax.axis_index("c")
        sid = lax.axis_index("s")
        wid = sid * NC + cid
        zeros16 = jnp.zeros((L,), jnp.float32)

        def zbody(i, _):
            hist_v[pl.ds(i * L, L)] = zeros16
            return 0
        lax.fori_loop(0, n_pad // L, zbody, 0)

        pltpu.sync_copy(col_hbm.at[pl.ds(wid * epw, epw)], idx_v)
        ones16 = jnp.ones((L,), jnp.float32)

        def body(j, _):
            idx = idx_v[pl.ds(j * L, L)]
            plsc.addupdate_scatter(hist_v, [idx], ones16)
            return 0
        lax.fori_loop(0, epw // L, body, 0)
        pltpu.sync_copy(hist_v, out_hbm.at[wid])

    PG = 3
    PI = 6
    assert nch % PI == 0 and PI % PG == 0

    @functools.partial(
        pl.kernel,
        out_type=jax.ShapeDtypeStruct((NC, n_pad, d), jnp.float32),
        mesh=mesh,
        scratch_types=[
            pltpu.VMEM_SHARED((n_pad, d), jnp.float32),
            pltpu.VMEM((PI, ch), jnp.int32),
            pltpu.VMEM((PI, ch), jnp.int32),
            pltpu.VMEM((PG, ch, d), jnp.float32),
            pltpu.SemaphoreType.DMA((PG,)),
            pltpu.SemaphoreType.DMA((PI,)),
            pltpu.SemaphoreType.DMA((PI,)),
        ],
    )
    def agg_kernel(xs_hbm, row_hbm, col_hbm, out_hbm,
                   acc_sh, ri_v, ci_v, g_v, sem_g, sem_i, sem_c):
        cid = lax.axis_index("c")
        sid = lax.axis_index("s")
        wid = sid * NC + cid
        rbase = sid * rpt
        ebase = wid * epw
        pltpu.sync_copy(xs_hbm.at[pl.ds(rbase, rpt)],
                        acc_sh.at[pl.ds(rbase, rpt)])

        for j in range(PI):
            pltpu.async_copy(row_hbm.at[pl.ds(ebase + j * ch, ch)],
                             ri_v.at[j], sem_i.at[j])
            pltpu.async_copy(col_hbm.at[pl.ds(ebase + j * ch, ch)],
                             ci_v.at[j], sem_c.at[j])
        for b in range(PG):
            pltpu.make_async_copy(row_hbm.at[pl.ds(ebase, ch)],
                                  ri_v.at[b], sem_i.at[b]).wait()
            pltpu.async_copy(xs_hbm.at[ri_v.at[b]], g_v.at[b], sem_g.at[b])

        plsc.subcore_barrier()

        @pl.loop(0, nch, step=PI)
        def _(k):
            for b in range(PI):
                kk = k + b
                gb = b % PG
                ib2 = (b + PG) % PI
                pltpu.make_async_copy(xs_hbm.at[ri_v.at[b]], g_v.at[gb],
                                      sem_g.at[gb]).wait()
                pltpu.make_async_copy(col_hbm.at[pl.ds(ebase, ch)],
                                      ci_v.at[b], sem_c.at[b]).wait()
                pltpu.sync_copy(g_v.at[gb], acc_sh.at[ci_v.at[b]], add=True)

                @pl.when(kk + PI < nch)
                def _issue_idx():
                    pltpu.async_copy(
                        row_hbm.at[pl.ds(ebase + (kk + PI) * ch, ch)],
                        ri_v.at[b], sem_i.at[b])
                    pltpu.async_copy(
                        col_hbm.at[pl.ds(ebase + (kk + PI) * ch, ch)],
                        ci_v.at[b], sem_c.at[b])

                @pl.when(kk + PG < nch)
                def _issue_gather():
                    pltpu.make_async_copy(row_hbm.at[pl.ds(ebase, ch)],
                                          ri_v.at[ib2], sem_i.at[ib2]).wait()
                    pltpu.async_copy(xs_hbm.at[ri_v.at[ib2]], g_v.at[gb],
                                     sem_g.at[gb])

        plsc.subcore_barrier()
        pltpu.sync_copy(acc_sh.at[pl.ds(rbase, rpt)],
                        out_hbm.at[cid, pl.ds(rbase, rpt)])

    return hist_kernel, agg_kernel


def kernel(x, edge_index, weight):
    n, d = x.shape
    e = edge_index.shape[1]

    n_pad = ((n + 1 + 127) // 128) * 128
    ch = 120
    grp = NW * ch * 6
    epw = ((e + grp - 1) // grp) * (ch * 6)
    e_pad = epw * NW

    npad_ids = jnp.arange(e_pad - e, dtype=jnp.int32)
    row = jnp.concatenate([edge_index[0], npad_ids % n])
    col = jnp.concatenate([edge_index[1], n + npad_ids % (n_pad - n)])
    x_pad = jnp.pad(x, ((0, n_pad - n), (0, 0)))

    hist_kernel, agg_kernel = _make_sc_kernels(n_pad, d, epw, ch)
    hist = hist_kernel(col)

    def tc_a_body(hist_ref, x_ref, w_ref, xs_ref):
        deg = jnp.sum(hist_ref[...], axis=0) + 1.0
        dis = lax.rsqrt(deg)
        w = _oni_norm(w_ref[...])
        xs_ref[...] = dis[:, None] * jnp.dot(
            x_ref[...], w, preferred_element_type=jnp.float32)

    xs = pl.pallas_call(
        tc_a_body,
        out_shape=jax.ShapeDtypeStruct((n_pad, d), jnp.float32),
    )(hist, x_pad, weight)

    parts = agg_kernel(xs, row, col)

    def tc_f_body(hist_ref, p_ref, xs_ref, out_ref):
        deg = jnp.sum(hist_ref[...], axis=0) + 1.0
        dis = lax.rsqrt(deg)
        res = dis[:, None] * (p_ref[0] + p_ref[1] - xs_ref[...])
        out_ref[...] = res[:out_ref.shape[0], :]

    out = pl.pallas_call(
        tc_f_body,
        out_shape=jax.ShapeDtypeStruct((n, d), jnp.float32),
    )(hist, parts, xs)
    return out


# speedup vs baseline: 3.6374x; 1.0075x over previous
import functools

import jax
import jax.numpy as jnp
from jax import lax
from jax.experimental import pallas as pl
from jax.experimental.pallas import tpu as pltpu
from jax.experimental.pallas import tpu_sc as plsc

EPS = 1e-05
T_ITERS = 4
NC = 2
NS = 16
NW = NC * NS
L = 16


def _oni_norm(w):
    d = w.shape[0]
    zc = w - jnp.mean(w, axis=1, keepdims=True)
    s = lax.dot_general(zc, zc, (((1,), (1,)), ((), ())),
                        preferred_element_type=jnp.float32)
    eye = jnp.eye(d, dtype=jnp.float32)
    s = s + EPS * eye
    norm_s = jnp.sqrt(jnp.sum(s * s))
    s = s / norm_s
    b = eye
    for _ in range(T_ITERS):
        b2 = jnp.dot(b, b, preferred_element_type=jnp.float32)
        b3 = jnp.dot(b2, b, preferred_element_type=jnp.float32)
        b = 1.5 * b - 0.5 * jnp.dot(b3, s, preferred_element_type=jnp.float32)
    return jnp.dot(b, zc, preferred_element_type=jnp.float32) / jnp.sqrt(norm_s)


def _make_sc_kernels(n_pad, d, epw, ch):
    mesh = plsc.VectorSubcoreMesh(core_axis_name="c", subcore_axis_name="s")
    rpt = n_pad // NS
    nch = epw // ch

    @functools.partial(
        pl.kernel,
        out_type=jax.ShapeDtypeStruct((NW, n_pad), jnp.float32),
        mesh=mesh,
        scratch_types=[
            pltpu.VMEM((n_pad,), jnp.float32),
            pltpu.VMEM((epw,), jnp.int32),
        ],
        compiler_params=pltpu.CompilerParams(needs_layout_passes=False),
    )
    def hist_kernel(col_hbm, out_hbm, hist_v, idx_v):
        ehw = col_hbm.shape[0] // NW
        cid = lax.axis_index("c")
        sid = lax.axis_index("s")
        wid = sid * NC + cid
        zeros16 = jnp.zeros((L,), jnp.float32)

        def zbody(i, _):
            hist_v[pl.ds(i * L, L)] = zeros16
            return 0
        lax.fori_loop(0, n_pad // L, zbody, 0)

        pltpu.sync_copy(col_hbm.at[pl.ds(wid * ehw, ehw)],
                        idx_v.at[pl.ds(0, ehw)])
        ones16 = jnp.ones((L,), jnp.float32)

        def body(j, _):
            idx = idx_v[pl.ds(j * L, L)]
            plsc.addupdate_scatter(hist_v, [idx], ones16)
            return 0
        lax.fori_loop(0, ehw // L, body, 0)
        pltpu.sync_copy(hist_v, out_hbm.at[wid])

    PG = 3
    PI = 6
    assert nch % PI == 0 and PI % PG == 0

    @functools.partial(
        pl.kernel,
        out_type=jax.ShapeDtypeStruct((NC, n_pad, d), jnp.float32),
        mesh=mesh,
        scratch_types=[
            pltpu.VMEM_SHARED((n_pad, d), jnp.float32),
            pltpu.VMEM((PI, ch), jnp.int32),
            pltpu.VMEM((PI, ch), jnp.int32),
            pltpu.VMEM((PG, ch, d), jnp.float32),
            pltpu.SemaphoreType.DMA((PG,)),
            pltpu.SemaphoreType.DMA((PI,)),
            pltpu.SemaphoreType.DMA((PI,)),
        ],
    )
    def agg_kernel(xs_hbm, row_hbm, col_hbm, out_hbm,
                   acc_sh, ri_v, ci_v, g_v, sem_g, sem_i, sem_c):
        cid = lax.axis_index("c")
        sid = lax.axis_index("s")
        wid = sid * NC + cid
        rbase = sid * rpt
        ebase = wid * epw
        pltpu.sync_copy(xs_hbm.at[pl.ds(rbase, rpt)],
                        acc_sh.at[pl.ds(rbase, rpt)])

        for j in range(PI):
            pltpu.async_copy(row_hbm.at[pl.ds(ebase + j * ch, ch)],
                             ri_v.at[j], sem_i.at[j])
            pltpu.async_copy(col_hbm.at[pl.ds(ebase + j * ch, ch)],
                             ci_v.at[j], sem_c.at[j])
        for b in range(PG):
            pltpu.make_async_copy(row_hbm.at[pl.ds(ebase, ch)],
                                  ri_v.at[b], sem_i.at[b]).wait()
            pltpu.async_copy(xs_hbm.at[ri_v.at[b]], g_v.at[b], sem_g.at[b])

        plsc.subcore_barrier()

        @pl.loop(0, nch, step=PI)
        def _(k):
            for b in range(PI):
                kk = k + b
                gb = b % PG
                ib2 = (b + PG) % PI
                pltpu.make_async_copy(xs_hbm.at[ri_v.at[b]], g_v.at[gb],
                                      sem_g.at[gb]).wait()
                pltpu.make_async_copy(col_hbm.at[pl.ds(ebase, ch)],
                                      ci_v.at[b], sem_c.at[b]).wait()
                pltpu.sync_copy(g_v.at[gb], acc_sh.at[ci_v.at[b]], add=True)

                @pl.when(kk + PI < nch)
                def _issue_idx():
                    pltpu.async_copy(
                        row_hbm.at[pl.ds(ebase + (kk + PI) * ch, ch)],
                        ri_v.at[b], sem_i.at[b])
                    pltpu.async_copy(
                        col_hbm.at[pl.ds(ebase + (kk + PI) * ch, ch)],
                        ci_v.at[b], sem_c.at[b])

                @pl.when(kk + PG < nch)
                def _issue_gather():
                    pltpu.make_async_copy(row_hbm.at[pl.ds(ebase, ch)],
                                          ri_v.at[ib2], sem_i.at[ib2]).wait()
                    pltpu.async_copy(xs_hbm.at[ri_v.at[ib2]], g_v.at[gb],
                                     sem_g.at[gb])

        plsc.subcore_barrier()
        pltpu.sync_copy(acc_sh.at[pl.ds(rbase, rpt)],
                        out_hbm.at[cid, pl.ds(rbase, rpt)])

    return hist_kernel, agg_kernel


def kernel(x, edge_index, weight):
    n, d = x.shape
    e = edge_index.shape[1]

    n_pad = ((n + 1 + 127) // 128) * 128
    ch = 120
    grp = NW * ch * 6
    epw = ((e + grp - 1) // grp) * (ch * 6)
    e_pad = epw * NW

    npad_ids = jnp.arange(e_pad - e, dtype=jnp.int32)
    row = jnp.concatenate([edge_index[0], npad_ids % n])
    col = jnp.concatenate([edge_index[1], n + npad_ids % (n_pad - n)])

    hist_kernel, agg_kernel = _make_sc_kernels(n_pad, d, epw, ch)
    hist = hist_kernel(edge_index[1])

    def tc_a_body(hist_ref, x_ref, w_ref, xs_ref):
        deg = jnp.sum(hist_ref[pl.ds(0, NW), pl.ds(0, n)], axis=0) + 1.0
        dis = lax.rsqrt(deg)
        w = _oni_norm(w_ref[...])
        xs_ref[pl.ds(0, n), :] = dis[:, None] * jnp.dot(
            x_ref[...], w, preferred_element_type=jnp.float32)

    xs = pl.pallas_call(
        tc_a_body,
        out_shape=jax.ShapeDtypeStruct((n_pad, d), jnp.float32),
    )(hist, x, weight)

    parts = agg_kernel(xs, row, col)

    def tc_f_body(hist_ref, p_ref, xs_ref, out_ref):
        deg = jnp.sum(hist_ref[...], axis=0) + 1.0
        dis = lax.rsqrt(deg)
        res = dis[:, None] * (p_ref[0] + p_ref[1] - xs_ref[...])
        out_ref[...] = res[:out_ref.shape[0], :]

    out = pl.pallas_call(
        tc_f_body,
        out_shape=jax.ShapeDtypeStruct((n, d), jnp.float32),
    )(hist, parts, xs)
    return out


# speedup vs baseline: 3.9078x; 1.0744x over previous
import functools

import jax
import jax.numpy as jnp
from jax import lax
from jax.experimental import pallas as pl
from jax.experimental.pallas import tpu as pltpu
from jax.experimental.pallas import tpu_sc as plsc

EPS = 1e-05
T_ITERS = 4
NC = 2
NS = 16
NW = NC * NS
L = 16


def _oni_norm(w):
    d = w.shape[0]
    zc = w - jnp.mean(w, axis=1, keepdims=True)
    s = lax.dot_general(zc, zc, (((1,), (1,)), ((), ())),
                        preferred_element_type=jnp.float32)
    eye = jnp.eye(d, dtype=jnp.float32)
    s = s + EPS * eye
    norm_s = jnp.sqrt(jnp.sum(s * s))
    s = s / norm_s
    b = eye
    for _ in range(T_ITERS):
        b2 = jnp.dot(b, b, preferred_element_type=jnp.float32)
        b3 = jnp.dot(b2, b, preferred_element_type=jnp.float32)
        b = 1.5 * b - 0.5 * jnp.dot(b3, s, preferred_element_type=jnp.float32)
    return jnp.dot(b, zc, preferred_element_type=jnp.float32) / jnp.sqrt(norm_s)


def _make_sc_kernels(n_pad, d, epw, ch):
    mesh = plsc.VectorSubcoreMesh(core_axis_name="c", subcore_axis_name="s")
    rpt = n_pad // NS
    nch = epw // ch
    tl = epw - nch * ch

    @functools.partial(
        pl.kernel,
        out_type=jax.ShapeDtypeStruct((NW, n_pad), jnp.float32),
        mesh=mesh,
        scratch_types=[
            pltpu.VMEM((n_pad,), jnp.float32),
            pltpu.VMEM((epw,), jnp.int32),
        ],
        compiler_params=pltpu.CompilerParams(needs_layout_passes=False),
    )
    def hist_kernel(ei_hbm, out_hbm, hist_v, idx_v):
        eoff = ei_hbm.shape[0] // 2
        cid = lax.axis_index("c")
        sid = lax.axis_index("s")
        wid = sid * NC + cid
        zeros16 = jnp.zeros((L,), jnp.float32)

        def zbody(i, _):
            hist_v[pl.ds(i * L, L)] = zeros16
            return 0
        lax.fori_loop(0, n_pad // L, zbody, 0)

        pltpu.sync_copy(ei_hbm.at[pl.ds(eoff + wid * epw, epw)], idx_v)
        ones16 = jnp.ones((L,), jnp.float32)

        def body(j, _):
            idx = idx_v[pl.ds(j * L, L)]
            plsc.addupdate_scatter(hist_v, [idx], ones16)
            return 0
        lax.fori_loop(0, epw // L, body, 0)
        pltpu.sync_copy(hist_v, out_hbm.at[wid])

    PG = 3
    PI = 6
    assert nch % PI == 0 and PI % PG == 0 and ch % 8 == 0 and tl % 8 == 0

    @functools.partial(
        pl.kernel,
        out_type=jax.ShapeDtypeStruct((NC, n_pad, d), jnp.float32),
        mesh=mesh,
        scratch_types=[
            pltpu.VMEM_SHARED((n_pad, d), jnp.float32),
            pltpu.VMEM((PI, ch), jnp.int32),
            pltpu.VMEM((PI, ch), jnp.int32),
            pltpu.VMEM((PG, ch, d), jnp.float32),
            pltpu.VMEM((1, tl), jnp.int32),
            pltpu.VMEM((1, tl), jnp.int32),
            pltpu.VMEM((tl, d), jnp.float32),
            pltpu.SemaphoreType.DMA((PG,)),
            pltpu.SemaphoreType.DMA((PI,)),
            pltpu.SemaphoreType.DMA((PI,)),
            pltpu.SemaphoreType.DMA,
        ],
    )
    def agg_kernel(xs_hbm, ei_hbm, out_hbm,
                   acc_sh, ri_v, ci_v, g_v, rt_v, ct_v, gt_v,
                   sem_g, sem_i, sem_c, sem_t):
        eoff = ei_hbm.shape[0] // 2
        cid = lax.axis_index("c")
        sid = lax.axis_index("s")
        wid = sid * NC + cid
        rbase = sid * rpt
        ebase = wid * epw
        pltpu.sync_copy(xs_hbm.at[pl.ds(rbase, rpt)],
                        acc_sh.at[pl.ds(rbase, rpt)])

        toff = ebase + nch * ch
        pltpu.sync_copy(ei_hbm.at[pl.ds(toff, tl)], rt_v.at[0])
        pltpu.sync_copy(ei_hbm.at[pl.ds(eoff + toff, tl)], ct_v.at[0])
        pltpu.async_copy(xs_hbm.at[rt_v.at[0]], gt_v, sem_t).wait()

        for j in range(PI):
            pltpu.async_copy(ei_hbm.at[pl.ds(ebase + j * ch, ch)],
                             ri_v.at[j], sem_i.at[j])
            pltpu.async_copy(ei_hbm.at[pl.ds(eoff + ebase + j * ch, ch)],
                             ci_v.at[j], sem_c.at[j])
        for b in range(PG):
            pltpu.make_async_copy(ei_hbm.at[pl.ds(ebase, ch)],
                                  ri_v.at[b], sem_i.at[b]).wait()
            pltpu.async_copy(xs_hbm.at[ri_v.at[b]], g_v.at[b], sem_g.at[b])

        plsc.subcore_barrier()
        pltpu.sync_copy(gt_v, acc_sh.at[ct_v.at[0]], add=True)

        @pl.loop(0, nch, step=PI)
        def _(k):
            for b in range(PI):
                kk = k + b
                gb = b % PG
                ib2 = (b + PG) % PI
                pltpu.make_async_copy(xs_hbm.at[ri_v.at[b]], g_v.at[gb],
                                      sem_g.at[gb]).wait()
                pltpu.make_async_copy(ei_hbm.at[pl.ds(eoff + ebase, ch)],
                                      ci_v.at[b], sem_c.at[b]).wait()
                pltpu.sync_copy(g_v.at[gb], acc_sh.at[ci_v.at[b]], add=True)

                @pl.when(kk + PI < nch)
                def _issue_idx():
                    pltpu.async_copy(
                        ei_hbm.at[pl.ds(ebase + (kk + PI) * ch, ch)],
                        ri_v.at[b], sem_i.at[b])
                    pltpu.async_copy(
                        ei_hbm.at[pl.ds(eoff + ebase + (kk + PI) * ch, ch)],
                        ci_v.at[b], sem_c.at[b])

                @pl.when(kk + PG < nch)
                def _issue_gather():
                    pltpu.make_async_copy(ei_hbm.at[pl.ds(ebase, ch)],
                                          ri_v.at[ib2], sem_i.at[ib2]).wait()
                    pltpu.async_copy(xs_hbm.at[ri_v.at[ib2]], g_v.at[gb],
                                     sem_g.at[gb])

        plsc.subcore_barrier()
        pltpu.sync_copy(acc_sh.at[pl.ds(rbase, rpt)],
                        out_hbm.at[cid, pl.ds(rbase, rpt)])

    return hist_kernel, agg_kernel


def kernel(x, edge_index, weight):
    n, d = x.shape
    e = edge_index.shape[1]

    n_pad = ((n + 1 + 127) // 128) * 128
    epw = e // NW
    ch = 104

    hist_kernel, agg_kernel = _make_sc_kernels(n_pad, d, epw, ch)
    ei_flat = edge_index.reshape(2 * e)
    hist = hist_kernel(ei_flat)

    def tc_a_body(hist_ref, x_ref, w_ref, xs_ref, dis_ref):
        deg = jnp.sum(hist_ref[...], axis=0) + 1.0
        dis = lax.rsqrt(deg)
        dis_ref[...] = dis[None, :]
        w = _oni_norm(w_ref[...])
        xs_ref[pl.ds(0, n), :] = dis[:n][:, None] * jnp.dot(
            x_ref[...], w, preferred_element_type=jnp.float32)

    xs, dis2 = pl.pallas_call(
        tc_a_body,
        out_shape=(
            jax.ShapeDtypeStruct((n_pad, d), jnp.float32),
            jax.ShapeDtypeStruct((1, n_pad), jnp.float32),
        ),
    )(hist, x, weight)

    parts = agg_kernel(xs, ei_flat)

    def tc_f_body(dis_ref, p_ref, xs_ref, out_ref):
        dis = dis_ref[0, :]
        res = dis[:, None] * (p_ref[0] + p_ref[1] - xs_ref[...])
        out_ref[...] = res[:out_ref.shape[0], :]

    out = pl.pallas_call(
        tc_f_body,
        out_shape=jax.ShapeDtypeStruct((n, d), jnp.float32),
    )(dis2, parts, xs)
    return out
